# baseline (speedup 1.0000x reference)
import functools

import jax
import jax.numpy as jnp
from jax import lax
from jax.experimental import pallas as pl
from jax.experimental.pallas import tpu as pltpu
from jax.experimental.pallas import tpu_sc as plsc

N_NODES = 10000
N_EDGES = 320000
D = 128
NC = 2
NS = 16
NW = NC * NS
E_PER_W = N_EDGES // NW
CHUNK = 112
N_CHUNKS = 90
E_PER_W_PAD = N_CHUNKS * CHUNK
N_PAIRS = N_CHUNKS // 2
N_EDGE_PAD = E_PER_W_PAD - E_PER_W
TRASH0 = 10008
N_PAD = 10240
ROWS_PER_TILE = N_PAD // NS
HR = 80
HR_PT = HR // NS
BATCH = 80


def _sc_aggregate(h, src3, dst3, zsum):
    mesh = plsc.VectorSubcoreMesh(core_axis_name="c", subcore_axis_name="s")

    @functools.partial(
        pl.kernel,
        mesh=mesh,
        compiler_params=pltpu.CompilerParams(needs_layout_passes=False),
        out_type=[
            jax.ShapeDtypeStruct((NC * N_PAD, D), jnp.float32),
            jax.ShapeDtypeStruct((NC * NS * HR, D), jnp.float32),
        ],
        scratch_types=[
            pltpu.VMEM((CHUNK,), jnp.int32),
            pltpu.VMEM((CHUNK,), jnp.int32),
            pltpu.VMEM((CHUNK,), jnp.int32),
            pltpu.VMEM((CHUNK,), jnp.int32),
            pltpu.VMEM((CHUNK, D), jnp.float32),
            pltpu.VMEM((CHUNK, D), jnp.float32),
            pltpu.VMEM((HR, D), jnp.float32),
            pltpu.VMEM_SHARED((N_PAD, D), jnp.float32),
            pltpu.SemaphoreType.DMA,
            pltpu.SemaphoreType.DMA,
            pltpu.SemaphoreType.DMA,
            pltpu.SemaphoreType.DMA,
            pltpu.SemaphoreType.DMA,
            pltpu.SemaphoreType.DMA,
        ],
    )
    def agg(h_hbm, src_hbm, dst_hbm, zsum_hbm,
            outs_hbm, outd_hbm,
            s0, s1, d0, d1, rows0, rows1, hist, sum_sh,
            sem0, sem1, semi0, semi1, semd0, semd1):
        c = lax.axis_index("c")
        s = lax.axis_index("s")
        wid = s * NC + c

        r0 = s * ROWS_PER_TILE
        my_acc = sum_sh.at[pl.ds(r0, ROWS_PER_TILE)]
        zslice = zsum_hbm.at[pl.ds(r0, ROWS_PER_TILE)]

        pltpu.sync_copy(src_hbm.at[wid, 0], s0)
        pltpu.sync_copy(src_hbm.at[wid, 1], s1)
        pltpu.async_copy(dst_hbm.at[wid, 0], d0, semd0)
        pltpu.async_copy(dst_hbm.at[wid, 1], d1, semd1)
        pltpu.sync_copy(zsum_hbm.at[pl.ds(0, HR)], hist)

        pltpu.sync_copy(zslice, my_acc)
        plsc.subcore_barrier()

        def hist_chunk(dbuf):
            for j in range(CHUNK // 16):
                iv = dbuf[pl.ds(j * 16, 16)]
                cnt, last = plsc.scan_count(iv)
                plsc.addupdate_scatter(
                    hist,
                    [lax.shift_right_logical(iv, 7),
                     lax.bitwise_and(iv, 127)],
                    cnt.astype(jnp.float32), mask=last)

        pltpu.async_copy(h_hbm.at[s0], rows0, sem0)
        pltpu.async_copy(h_hbm.at[s1], rows1, sem1)

        def pair(k, carry):
            i0 = k * 2
            nx0 = lax.min(i0 + 2, N_CHUNKS - 1)
            nx1 = lax.min(i0 + 3, N_CHUNKS - 1)
            pltpu.make_async_copy(h_hbm.at[s0], rows0, sem0).wait()
            pltpu.async_copy(src_hbm.at[wid, nx0], s0, semi0)
            pltpu.make_async_copy(dst_hbm.at[wid, i0], d0, semd0).wait()
            hist_chunk(d0)
            pltpu.sync_copy(rows0, sum_sh.at[d0], add=True)
            pltpu.async_copy(dst_hbm.at[wid, nx0], d0, semd0)
            pltpu.make_async_copy(src_hbm.at[wid, nx0], s0, semi0).wait()
            pltpu.async_copy(h_hbm.at[s0], rows0, sem0)
            pltpu.make_async_copy(h_hbm.at[s1], rows1, sem1).wait()
            pltpu.async_copy(src_hbm.at[wid, nx1], s1, semi1)
            pltpu.make_async_copy(dst_hbm.at[wid, i0 + 1], d1, semd1).wait()
            hist_chunk(d1)
            pltpu.sync_copy(rows1, sum_sh.at[d1], add=True)
            pltpu.async_copy(dst_hbm.at[wid, nx1], d1, semd1)
            pltpu.make_async_copy(src_hbm.at[wid, nx1], s1, semi1).wait()
            pltpu.async_copy(h_hbm.at[s1], rows1, sem1)
            return carry

        lax.fori_loop(0, N_PAIRS, pair, 0)
        last_c = N_CHUNKS - 1
        pltpu.make_async_copy(h_hbm.at[s0], rows0, sem0).wait()
        pltpu.make_async_copy(h_hbm.at[s1], rows1, sem1).wait()
        pltpu.make_async_copy(dst_hbm.at[wid, last_c], d0, semd0).wait()
        pltpu.make_async_copy(dst_hbm.at[wid, last_c], d1, semd1).wait()
        plsc.subcore_barrier()

        out_r0 = c * N_PAD + r0
        pltpu.sync_copy(my_acc, outs_hbm.at[pl.ds(out_r0, ROWS_PER_TILE)])
        pltpu.sync_copy(hist, outd_hbm.at[pl.ds((c * NS + s) * HR, HR)])

    return agg(h, src3, dst3, zsum)


def _tc_body(h_ref, ps_ref, pd_ref, ws_ref, wn_ref, bs_ref, g_ref, be_ref,
             o_ref):
    summed = ps_ref[0:N_NODES, :] + ps_ref[N_PAD:N_PAD + N_NODES, :]
    packed = pd_ref[0:HR, :]
    for w in range(1, NC * NS):
        packed = packed + pd_ref[w * HR:(w + 1) * HR, :]
    rowsel = (lax.broadcasted_iota(jnp.int32, (N_NODES, HR), 0) // D
              == lax.broadcasted_iota(jnp.int32, (N_NODES, HR), 1)
              ).astype(jnp.float32)
    lanesel = (lax.broadcasted_iota(jnp.int32, (N_NODES, D), 0) % D
               == lax.broadcasted_iota(jnp.int32, (N_NODES, D), 1)
               ).astype(jnp.float32)
    q = lax.dot_general(rowsel, packed, (((1,), (0,)), ((), ())),
                        precision=lax.Precision.HIGHEST,
                        preferred_element_type=jnp.float32)
    deg = jnp.sum(q * lanesel, axis=1, keepdims=True)
    neigh = summed / jnp.maximum(deg, 1.0)
    dn = (((1,), (1,)), ((), ()))
    out = (lax.dot_general(h_ref[...], ws_ref[...], dn,
                           preferred_element_type=jnp.float32)
           + lax.dot_general(neigh, wn_ref[...], dn,
                             preferred_element_type=jnp.float32)
           + bs_ref[...])
    mean = jnp.mean(out, axis=0, keepdims=True)
    var = jnp.mean(jnp.square(out - mean), axis=0, keepdims=True)
    o_ref[...] = jnp.maximum(
        g_ref[...] * (out - mean) * lax.rsqrt(var + 1e-5) + be_ref[...], 0.0)


def kernel(h, edge_index, W_self, b_self, W_neigh, gamma, beta):
    src2 = edge_index[0].astype(jnp.int32).reshape(NW, E_PER_W)
    dst2 = edge_index[1].astype(jnp.int32).reshape(NW, E_PER_W)
    spad = jnp.zeros((NW, N_EDGE_PAD), jnp.int32)
    dpad = jnp.broadcast_to(TRASH0 + jnp.arange(N_EDGE_PAD, dtype=jnp.int32),
                            (NW, N_EDGE_PAD))
    src = jnp.concatenate([src2, spad], axis=1).reshape(NW, N_CHUNKS, CHUNK)
    dst = jnp.concatenate([dst2, dpad], axis=1).reshape(NW, N_CHUNKS, CHUNK)
    zsum = jnp.zeros((N_PAD, D), jnp.float32)

    psum, pdeg = _sc_aggregate(h, src, dst, zsum)

    out = pl.pallas_call(
        _tc_body,
        out_shape=jax.ShapeDtypeStruct((N_NODES, D), jnp.float32),
    )(h, psum, pdeg, W_self, W_neigh,
      b_self.reshape(1, D), gamma.reshape(1, D), beta.reshape(1, D))
    return out


# speedup vs baseline: 2.5177x; 2.5177x over previous
import functools

import jax
import jax.numpy as jnp
from jax import lax
from jax.experimental import pallas as pl
from jax.experimental.pallas import tpu as pltpu
from jax.experimental.pallas import tpu_sc as plsc

N_NODES = 10000
N_EDGES = 320000
D = 128
NC = 2
NS = 16
NW = NC * NS
E_PER_W = N_EDGES // NW
CHUNK = 80
N_CHUNKS = E_PER_W // CHUNK
N_PAIRS = (N_CHUNKS - 1) // 2
N_PAD = 10240
ROWS_PER_TILE = N_PAD // NS
HR = 80


def _sc_aggregate(h, src3, dst3, zsum):
    mesh = plsc.VectorSubcoreMesh(core_axis_name="c", subcore_axis_name="s")

    @functools.partial(
        pl.kernel,
        mesh=mesh,
        compiler_params=pltpu.CompilerParams(needs_layout_passes=False),
        out_type=[
            jax.ShapeDtypeStruct((NC * N_PAD, D), jnp.float32),
            jax.ShapeDtypeStruct((NC * NS * HR, D), jnp.float32),
        ],
        scratch_types=[
            pltpu.VMEM((CHUNK,), jnp.int32),
            pltpu.VMEM((CHUNK,), jnp.int32),
            pltpu.VMEM((N_CHUNKS, CHUNK), jnp.int32),
            pltpu.VMEM((CHUNK, D), jnp.float32),
            pltpu.VMEM((CHUNK, D), jnp.float32),
            pltpu.VMEM((HR, D), jnp.float32),
            pltpu.VMEM_SHARED((N_PAD, D), jnp.float32),
            pltpu.SemaphoreType.DMA,
            pltpu.SemaphoreType.DMA,
            pltpu.SemaphoreType.DMA,
            pltpu.SemaphoreType.DMA,
        ],
    )
    def agg(h_hbm, src_hbm, dst_hbm, zsum_hbm,
            outs_hbm, outd_hbm,
            s0, s1, idx_d, rows0, rows1, hist, sum_sh,
            sem0, sem1, semi0, semi1):
        c = lax.axis_index("c")
        s = lax.axis_index("s")
        wid = s * NC + c

        r0 = s * ROWS_PER_TILE
        my_acc = sum_sh.at[pl.ds(r0, ROWS_PER_TILE)]
        zslice = zsum_hbm.at[pl.ds(r0, ROWS_PER_TILE)]

        pltpu.sync_copy(dst_hbm.at[wid], idx_d)
        pltpu.sync_copy(src_hbm.at[wid, 0], s0)
        pltpu.sync_copy(src_hbm.at[wid, 1], s1)
        pltpu.sync_copy(zsum_hbm.at[pl.ds(0, HR)], hist)

        pltpu.sync_copy(zslice, my_acc)
        plsc.subcore_barrier()

        def hist_chunk(i):
            for j in range(CHUNK // 16):
                iv = idx_d[i, pl.ds(j * 16, 16)]
                cnt, last = plsc.scan_count(iv)
                plsc.addupdate_scatter(
                    hist,
                    [lax.shift_right_logical(iv, 7),
                     lax.bitwise_and(iv, 127)],
                    cnt.astype(jnp.float32), mask=last)

        pltpu.async_copy(h_hbm.at[s0], rows0, sem0)
        pltpu.async_copy(h_hbm.at[s1], rows1, sem1)

        def pair(k, carry):
            i0 = k * 2
            pltpu.make_async_copy(h_hbm.at[s0], rows0, sem0).wait()
            pltpu.async_copy(src_hbm.at[wid, i0 + 2], s0, semi0)
            hist_chunk(i0)
            pltpu.sync_copy(rows0, sum_sh.at[idx_d.at[i0]], add=True)
            pltpu.make_async_copy(src_hbm.at[wid, i0 + 2], s0, semi0).wait()
            pltpu.async_copy(h_hbm.at[s0], rows0, sem0)
            pltpu.make_async_copy(h_hbm.at[s1], rows1, sem1).wait()
            nxt = lax.min(i0 + 3, N_CHUNKS - 1)
            pltpu.async_copy(src_hbm.at[wid, nxt], s1, semi1)
            hist_chunk(i0 + 1)
            pltpu.sync_copy(rows1, sum_sh.at[idx_d.at[i0 + 1]], add=True)
            pltpu.make_async_copy(src_hbm.at[wid, nxt], s1, semi1).wait()
            pltpu.async_copy(h_hbm.at[s1], rows1, sem1)
            return carry

        lax.fori_loop(0, N_PAIRS, pair, 0)
        last_c = N_CHUNKS - 1
        pltpu.make_async_copy(h_hbm.at[s0], rows0, sem0).wait()
        hist_chunk(last_c)
        pltpu.sync_copy(rows0, sum_sh.at[idx_d.at[last_c]], add=True)
        pltpu.make_async_copy(h_hbm.at[s1], rows1, sem1).wait()
        plsc.subcore_barrier()

        out_r0 = c * N_PAD + r0
        pltpu.sync_copy(my_acc, outs_hbm.at[pl.ds(out_r0, ROWS_PER_TILE)])
        pltpu.sync_copy(hist, outd_hbm.at[pl.ds((c * NS + s) * HR, HR)])

    return agg(h, src3, dst3, zsum)


def _tc_body(h_ref, ps_ref, pd_ref, ws_ref, wn_ref, bs_ref, g_ref, be_ref,
             o_ref):
    summed = ps_ref[0:N_NODES, :] + ps_ref[N_PAD:N_PAD + N_NODES, :]
    packed = pd_ref[0:HR, :]
    for w in range(1, NC * NS):
        packed = packed + pd_ref[w * HR:(w + 1) * HR, :]
    rowsel = (lax.broadcasted_iota(jnp.int32, (N_NODES, HR), 0) // D
              == lax.broadcasted_iota(jnp.int32, (N_NODES, HR), 1)
              ).astype(jnp.float32)
    lanesel = (lax.broadcasted_iota(jnp.int32, (N_NODES, D), 0) % D
               == lax.broadcasted_iota(jnp.int32, (N_NODES, D), 1)
               ).astype(jnp.float32)
    q = lax.dot_general(rowsel, packed, (((1,), (0,)), ((), ())),
                        precision=lax.Precision.HIGHEST,
                        preferred_element_type=jnp.float32)
    deg = jnp.sum(q * lanesel, axis=1, keepdims=True)
    neigh = summed / jnp.maximum(deg, 1.0)
    dn = (((1,), (1,)), ((), ()))
    out = (lax.dot_general(h_ref[...], ws_ref[...], dn,
                           preferred_element_type=jnp.float32)
           + lax.dot_general(neigh, wn_ref[...], dn,
                             preferred_element_type=jnp.float32)
           + bs_ref[...])
    mean = jnp.mean(out, axis=0, keepdims=True)
    var = jnp.mean(jnp.square(out - mean), axis=0, keepdims=True)
    o_ref[...] = jnp.maximum(
        g_ref[...] * (out - mean) * lax.rsqrt(var + 1e-5) + be_ref[...], 0.0)


def kernel(h, edge_index, W_self, b_self, W_neigh, gamma, beta):
    src = edge_index[0].astype(jnp.int32).reshape(NW, N_CHUNKS, CHUNK)
    dst = edge_index[1].astype(jnp.int32).reshape(NW, N_CHUNKS, CHUNK)
    zsum = jnp.zeros((N_PAD, D), jnp.float32)

    psum, pdeg = _sc_aggregate(h, src, dst, zsum)

    out = pl.pallas_call(
        _tc_body,
        out_shape=jax.ShapeDtypeStruct((N_NODES, D), jnp.float32),
    )(h, psum, pdeg, W_self, W_neigh,
      b_self.reshape(1, D), gamma.reshape(1, D), beta.reshape(1, D))
    return out
